# baseline (speedup 1.0000x reference)
import functools

import jax
import jax.numpy as jnp
from jax import lax
from jax.experimental import pallas as pl
from jax.experimental.pallas import tpu as pltpu
from jax.experimental.pallas import tpu_sc as plsc

NS = 26
ND = 13
VOCAB = 100000
ED = 16
OUT_D = NS * ED + 1
LANES = 16
R = 128


def _make_sc_kernel(B: int):
    NW = 32
    BPW = B // NW
    NCHUNK = BPW // R

    mesh = plsc.VectorSubcoreMesh(core_axis_name="c", subcore_axis_name="s")

    @functools.partial(
        pl.kernel,
        mesh=mesh,
        out_type=jax.ShapeDtypeStruct((B * OUT_D,), jnp.float32),
        scratch_types=[
            pltpu.VMEM((R * NS,), jnp.int32),
            pltpu.VMEM((NS, R), jnp.int32),
            pltpu.VMEM((NS, R, ED), jnp.float32),
            pltpu.VMEM((NS, R), jnp.float32),
            pltpu.VMEM((R * ND,), jnp.float32),
            pltpu.VMEM((LANES,), jnp.float32),
            pltpu.VMEM((R * OUT_D,), jnp.float32),
            pltpu.SemaphoreType.DMA,
        ],
        compiler_params=pltpu.CompilerParams(
            needs_layout_passes=False, use_tc_tiling_on_sc=False),
    )
    def sc_kernel(sp_hbm, dense_hbm, table_hbm, lin_hbm, w_hbm, out_hbm,
                  spbuf, idxbuf, gbuf, lbuf, dbuf, wbuf, obuf, sem):
        nc = 2
        wid = lax.axis_index("s") * nc + lax.axis_index("c")
        wbase = wid * BPW
        iota = jnp.arange(LANES, dtype=jnp.int32)

        pltpu.sync_copy(w_hbm, wbuf)
        wv = wbuf[...]

        def chunk_body(c, carry):
            rbase = wbase + c * R

            pltpu.sync_copy(sp_hbm.at[pl.ds(rbase * NS, R * NS)], spbuf)
            pltpu.sync_copy(dense_hbm.at[pl.ds(rbase * ND, R * ND)], dbuf)

            for f in range(NS):
                for g in range(R // LANES):
                    pos = iota * NS + (g * LANES * NS + f)
                    spv = plsc.load_gather(spbuf, [pos])
                    idxbuf[f, pl.ds(g * LANES, LANES)] = spv + f * VOCAB

            copies = []
            for f in range(NS):
                copies.append(
                    pltpu.async_copy(table_hbm.at[idxbuf.at[f]], gbuf.at[f], sem))
                copies.append(
                    pltpu.async_copy(lin_hbm.at[idxbuf.at[f]], lbuf.at[f], sem))
            for cp in copies:
                cp.wait()

            for g in range(R // LANES):
                acc = jnp.zeros((LANES,), jnp.float32)
                for f in range(NS):
                    acc = acc + lbuf[f, pl.ds(g * LANES, LANES)]
                for d in range(ND):
                    dv = plsc.load_gather(dbuf, [iota * ND + (g * LANES * ND + d)])
                    acc = acc + dv * wv[d]
                plsc.store_scatter(
                    obuf, [iota * OUT_D + (g * LANES * OUT_D + OUT_D - 1)], acc)

            def row_body(b, carry2):
                bvec = jnp.full((LANES,), b, jnp.int32)
                for f in range(NS):
                    v = plsc.load_gather(
                        gbuf, [jnp.full((LANES,), f, jnp.int32), bvec, iota])
                    obuf[pl.ds(b * OUT_D + f * ED, ED)] = v
                return carry2

            lax.fori_loop(0, R, row_body, 0)

            pltpu.sync_copy(obuf, out_hbm.at[pl.ds(rbase * OUT_D, R * OUT_D)])
            return carry

        lax.fori_loop(0, NCHUNK, chunk_body, 0)

    return sc_kernel


def kernel(sparse_indices, dense, table, lin_table, lin_dense_w):
    B = sparse_indices.shape[0]
    sp_flat = sparse_indices.astype(jnp.int32).reshape(-1)
    dense_flat = dense.reshape(-1)
    lin_flat = lin_table.reshape(-1)
    w_pad = jnp.zeros((LANES,), jnp.float32).at[:ND].set(lin_dense_w[0])
    out_flat = _make_sc_kernel(B)(sp_flat, dense_flat, table, lin_flat, w_pad)
    return out_flat.reshape(B, OUT_D)


# speedup vs baseline: 1.0083x; 1.0083x over previous
import functools

import jax
import jax.numpy as jnp
from jax import lax
from jax.experimental import pallas as pl
from jax.experimental.pallas import tpu as pltpu
from jax.experimental.pallas import tpu_sc as plsc

NS = 26
ND = 13
VOCAB = 100000
ED = 16
OUT_D = NS * ED + 1
LANES = 16
R = 128


def _make_sc_kernel(B: int):
    NW = 32
    BPW = B // NW
    NCHUNK = BPW // R

    mesh = plsc.VectorSubcoreMesh(core_axis_name="c", subcore_axis_name="s")

    @functools.partial(
        pl.kernel,
        mesh=mesh,
        out_type=jax.ShapeDtypeStruct((B * OUT_D,), jnp.float32),
        scratch_types=[
            pltpu.VMEM((NS, R), jnp.int32),
            pltpu.VMEM((NS, R), jnp.int32),
            pltpu.VMEM((NS, R, ED), jnp.float32),
            pltpu.VMEM((NS, R), jnp.float32),
            pltpu.VMEM((ND, R), jnp.float32),
            pltpu.VMEM((LANES,), jnp.float32),
            pltpu.VMEM((R * OUT_D,), jnp.float32),
            pltpu.SemaphoreType.DMA,
            pltpu.SemaphoreType.DMA,
        ],
        compiler_params=pltpu.CompilerParams(
            needs_layout_passes=False, use_tc_tiling_on_sc=False),
    )
    def sc_kernel(sp_hbm, dense_hbm, table_hbm, lin_hbm, w_hbm, out_hbm,
                  spbuf, idxbuf, gbuf, lbuf, dbuf, wbuf, obuf, sem_e, sem_l):
        nc = 2
        wid = lax.axis_index("s") * nc + lax.axis_index("c")
        wbase = wid * BPW
        iota = jnp.arange(LANES, dtype=jnp.int32)

        pltpu.sync_copy(w_hbm, wbuf)
        wv = wbuf[...]

        def chunk_body(c, carry):
            rbase = wbase + c * R

            pltpu.sync_copy(sp_hbm.at[:, pl.ds(rbase, R)], spbuf)
            pltpu.sync_copy(dense_hbm.at[:, pl.ds(rbase, R)], dbuf)

            for f in range(NS):
                for g in range(R // LANES):
                    sl = pl.ds(g * LANES, LANES)
                    idxbuf[f, sl] = spbuf[f, sl] + f * VOCAB

            emb_copies = []
            lin_copies = []
            for f in range(NS):
                emb_copies.append(
                    pltpu.async_copy(table_hbm.at[idxbuf.at[f]], gbuf.at[f],
                                     sem_e))
                lin_copies.append(
                    pltpu.async_copy(lin_hbm.at[idxbuf.at[f]], lbuf.at[f],
                                     sem_l))
            for cp in lin_copies:
                cp.wait()

            for g in range(R // LANES):
                sl = pl.ds(g * LANES, LANES)
                acc = jnp.zeros((LANES,), jnp.float32)
                for f in range(NS):
                    acc = acc + lbuf[f, sl]
                for d in range(ND):
                    acc = acc + dbuf[d, sl] * wv[d]
                plsc.store_scatter(
                    obuf, [iota * OUT_D + (g * LANES * OUT_D + OUT_D - 1)], acc)

            for cp in emb_copies:
                cp.wait()

            def row_body(b, carry2):
                bvec = jnp.full((LANES,), b, jnp.int32)
                for f in range(NS):
                    v = plsc.load_gather(
                        gbuf, [jnp.full((LANES,), f, jnp.int32), bvec, iota])
                    obuf[pl.ds(b * OUT_D + f * ED, ED)] = v
                return carry2

            lax.fori_loop(0, R, row_body, 0)

            pltpu.sync_copy(obuf, out_hbm.at[pl.ds(rbase * OUT_D, R * OUT_D)])
            return carry

        lax.fori_loop(0, NCHUNK, chunk_body, 0)

    return sc_kernel


def kernel(sparse_indices, dense, table, lin_table, lin_dense_w):
    B = sparse_indices.shape[0]
    sp_t = sparse_indices.astype(jnp.int32).T
    dense_t = dense.T
    lin_flat = lin_table.reshape(-1)
    w_pad = jnp.pad(lin_dense_w.reshape(-1), (0, LANES - ND))
    out_flat = _make_sc_kernel(B)(sp_t, dense_t, table, lin_flat, w_pad)
    return out_flat.reshape(B, OUT_D)
